# baseline (speedup 1.0000x reference)
import jax
from jax.experimental import pallas as pl
from jax.experimental.pallas import tpu as pltpu

_S_BLK = 1024
_NBUF = 4


def _dma_body(s, b, f, table_hbm, out_hbm, bufs, in_sems, out_sems):
    n = s // _S_BLK

    def in_copy(i):
        return pltpu.make_async_copy(
            table_hbm.at[pl.ds(i * _S_BLK, _S_BLK), :],
            bufs.at[i % _NBUF],
            in_sems.at[i % _NBUF],
        )

    def out_copy(i, j):
        return pltpu.make_async_copy(
            bufs.at[i % _NBUF],
            out_hbm.at[pl.ds(i * _S_BLK, _S_BLK), j, :],
            out_sems.at[i % _NBUF, j],
        )

    for i in range(min(_NBUF, n)):
        in_copy(i).start()
    for i in range(n):
        if i >= _NBUF:
            for j in range(b):
                out_copy(i - _NBUF, j).wait()
            in_copy(i).start()
        in_copy(i).wait()
        for j in range(b):
            out_copy(i, j).start()
    for i in range(max(0, n - _NBUF), n):
        for j in range(b):
            out_copy(i, j).wait()


def kernel(x, table):
    s, b, f = x.shape
    return pl.pallas_call(
        lambda t, o, bufs, isem, osem: _dma_body(s, b, f, t, o, bufs, isem, osem),
        in_specs=[pl.BlockSpec(memory_space=pltpu.MemorySpace.HBM)],
        out_specs=pl.BlockSpec(memory_space=pltpu.MemorySpace.HBM),
        out_shape=jax.ShapeDtypeStruct((s, b, f), table.dtype),
        scratch_shapes=[
            pltpu.VMEM((_NBUF, _S_BLK, f), table.dtype),
            pltpu.SemaphoreType.DMA((_NBUF,)),
            pltpu.SemaphoreType.DMA((_NBUF, b)),
        ],
    )(table)
